# Initial kernel scaffold; baseline (speedup 1.0000x reference)
import jax
import jax.numpy as jnp
from jax.experimental import pallas as pl


def kernel(x, edge_index, edge_attr, W_l, b_l, W_r, b_r, att, conv_bias, bn_gamma, bn_beta, W_lin, b_lin, a, b, W_dec, b_dec):
    raise NotImplementedError("write your pallas kernel here")


# speedup vs baseline: 17.5928x; 17.5928x over previous
import functools

import jax
import jax.numpy as jnp
from jax import lax
from jax.experimental import pallas as pl
from jax.experimental.pallas import tpu as pltpu
from jax.experimental.pallas import tpu_sc as plsc

NC = 2
NS = 16
NW = NC * NS
L = 16
B = 128
HEADS = 4
C = 16
HID = HEADS * C
ACC_W = 80


def _proj_body(x_ref, wl_ref, bl_ref, wr_ref, br_ref, xl_ref, xr_ref):
    x = x_ref[...]
    xl_ref[...] = x @ wl_ref[...] + bl_ref[...][None, :]
    xr_ref[...] = x @ wr_ref[...] + br_ref[...][None, :]


def _iota16():
    return lax.iota(jnp.int32, L)


def _msg_body(pb, nacc, xl_hbm, xr_hbm, srcs_hbm, dsts_hbm, att_hbm, part_hbm,
              idx_src, idx_dst, xls, xrd, val, att_v, zrow, acc, sem1, sem2):
    cid = lax.axis_index("c")
    sid = lax.axis_index("s")
    wid = cid * NS + sid
    rows_per_sub = nacc // NS

    zeros16 = jnp.zeros((L,), jnp.float32)
    for c16 in range(ACC_W // L):
        zrow[pl.ds(c16 * L, L)] = zeros16

    def _zero_val(r, carry):
        for c16 in range(ACC_W // L):
            val[r, pl.ds(c16 * L, L)] = zeros16
        return carry
    lax.fori_loop(0, B, _zero_val, 0)

    def _zero_acc(r, carry):
        pltpu.sync_copy(zrow, acc.at[sid * rows_per_sub + r])
        return carry
    lax.fori_loop(0, rows_per_sub, _zero_acc, 0)

    pltpu.sync_copy(srcs_hbm.at[wid], idx_src)
    pltpu.sync_copy(dsts_hbm.at[wid], idx_dst)
    pltpu.sync_copy(att_hbm, att_v)
    plsc.subcore_barrier()

    def _block(j, carry):
        d1 = pltpu.async_copy(xl_hbm.at[idx_src.at[j]], xls, sem1)
        d2 = pltpu.async_copy(xr_hbm.at[idx_dst.at[j]], xrd, sem2)
        d1.wait()
        d2.wait()

        att_rows = [att_v[h, :] for h in range(HEADS)]

        def _group(g, gcarry):
            row = g * L + _iota16()
            ws = []
            for h in range(HEADS):
                acc_h = jnp.zeros((L,), jnp.float32)
                for cc in range(C):
                    col = jnp.full((L,), h * C + cc, jnp.int32)
                    t = (plsc.load_gather(xls, [row, col])
                         + plsc.load_gather(xrd, [row, col]))
                    lk = jnp.maximum(t, 0.2 * t)
                    acc_h = acc_h + lk * att_rows[h][cc]
                ws.append(jnp.exp(acc_h))
            for h in range(HEADS):
                w_h = ws[h]
                for cc in range(C):
                    col = jnp.full((L,), h * C + cc, jnp.int32)
                    v = plsc.load_gather(xls, [row, col]) * w_h
                    plsc.store_scatter(val, [row, col], v)
                plsc.store_scatter(val, [row, jnp.full((L,), HID + h, jnp.int32)], w_h)
            return gcarry
        lax.fori_loop(0, B // L, _group, 0)

        pltpu.sync_copy(val, acc.at[idx_dst.at[j]], add=True)
        return carry
    lax.fori_loop(0, pb, _block, 0)

    plsc.subcore_barrier()
    pltpu.sync_copy(acc.at[pl.ds(sid * rows_per_sub, rows_per_sub)],
                    part_hbm.at[cid, pl.ds(sid * rows_per_sub, rows_per_sub)])


def _tail_body(n, part_ref, cb_ref, g_ref, be_ref, wlin_ref, blin_ref,
               wdec_ref, bdec_ref, z_ref, v2_ref):
    P = part_ref[0] + part_ref[1]
    val = P[:n, :HID]
    w16 = P[:n, HID:HID + L]
    r16 = lax.broadcasted_iota(jnp.int32, (L, HID), 0)
    c16 = lax.broadcasted_iota(jnp.int32, (L, HID), 1)
    S = (r16 == c16 // C).astype(jnp.float32)
    den = w16 @ S
    H1 = jnp.maximum(val / den + cb_ref[...][None, :], 0.0)
    mean = jnp.mean(H1, axis=0)
    var = jnp.mean((H1 - mean[None, :]) ** 2, axis=0)
    Hn = (H1 - mean[None, :]) / jnp.sqrt(var + 1e-5) * g_ref[...][None, :] + be_ref[...][None, :]
    z = Hn @ wlin_ref[...] + blin_ref[...][None, :]
    z_ref[...] = z
    v2_ref[...] = z @ wdec_ref[...] + bdec_ref[...][None, :]


def _dec_body(pb, z_hbm, srcs_hbm, dsts_hbm, ab_hbm, v1_hbm,
              idx_src, idx_dst, zs, zd, outb, ab_v, sem1, sem2):
    cid = lax.axis_index("c")
    sid = lax.axis_index("s")
    wid = cid * NS + sid
    pltpu.sync_copy(srcs_hbm.at[wid], idx_src)
    pltpu.sync_copy(dsts_hbm.at[wid], idx_dst)
    pltpu.sync_copy(ab_hbm, ab_v)
    abv = ab_v[...]
    ra = jnp.maximum(abv[0], 0.0)
    sb = abv[1]

    def _block(j, carry):
        d1 = pltpu.async_copy(z_hbm.at[idx_src.at[j]], zs, sem1)
        d2 = pltpu.async_copy(z_hbm.at[idx_dst.at[j]], zd, sem2)
        d1.wait()
        d2.wait()

        def _group(g, gcarry):
            row = g * L + _iota16()
            acc = jnp.zeros((L,), jnp.float32)
            for col_i in range(HID):
                col = jnp.full((L,), col_i, jnp.int32)
                d = (plsc.load_gather(zs, [row, col])
                     - plsc.load_gather(zd, [row, col]))
                acc = acc + d * d
            dist = ra * acc + sb
            outb[pl.ds(g * L, L)] = 1.0 / (1.0 + jnp.exp(dist))
            return gcarry
        lax.fori_loop(0, B // L, _group, 0)

        pltpu.sync_copy(outb, v1_hbm.at[pl.ds((wid * pb + j) * B, B)])
        return carry
    lax.fori_loop(0, pb, _block, 0)


def kernel(x, edge_index, edge_attr, W_l, b_l, W_r, b_r, att, conv_bias,
           bn_gamma, bn_beta, W_lin, b_lin, a, b, W_dec, b_dec):
    n = x.shape[0]
    e_orig = edge_index.shape[1]

    xl, xr = pl.pallas_call(
        _proj_body,
        out_shape=(jax.ShapeDtypeStruct((n, HID), jnp.float32),
                   jax.ShapeDtypeStruct((n, HID), jnp.float32)),
    )(x, W_l, b_l, W_r, b_r)

    loop = jnp.arange(n, dtype=edge_index.dtype)
    e2 = e_orig + n
    pb_msg = -(-e2 // (NW * B))
    e2_pad = pb_msg * NW * B
    src2 = jnp.concatenate([edge_index[0], loop,
                            jnp.zeros((e2_pad - e2,), jnp.int32)])
    dst2 = jnp.concatenate([edge_index[1], loop,
                            jnp.full((e2_pad - e2,), n, jnp.int32)])
    srcs = src2.reshape(NW, pb_msg, B)
    dsts = dst2.reshape(NW, pb_msg, B)
    nacc = -(-(n + 1) // (NS * 8)) * (NS * 8)

    mesh = plsc.VectorSubcoreMesh(core_axis_name="c", subcore_axis_name="s",
                                  num_cores=NC, num_subcores=NS)
    msg = functools.partial(
        pl.kernel,
        out_type=jax.ShapeDtypeStruct((NC, nacc, ACC_W), jnp.float32),
        mesh=mesh,
        compiler_params=pltpu.CompilerParams(needs_layout_passes=False, use_tc_tiling_on_sc=False),
        scratch_types=[
            pltpu.VMEM((pb_msg, B), jnp.int32),
            pltpu.VMEM((pb_msg, B), jnp.int32),
            pltpu.VMEM((B, HID), jnp.float32),
            pltpu.VMEM((B, HID), jnp.float32),
            pltpu.VMEM((B, ACC_W), jnp.float32),
            pltpu.VMEM((HEADS, C), jnp.float32),
            pltpu.VMEM((ACC_W,), jnp.float32),
            pltpu.VMEM_SHARED((nacc, ACC_W), jnp.float32),
            pltpu.SemaphoreType.DMA,
            pltpu.SemaphoreType.DMA,
        ],
    )(functools.partial(_msg_body, pb_msg, nacc))
    partials = msg(xl, xr, srcs, dsts, att)

    z, value2 = pl.pallas_call(
        functools.partial(_tail_body, n),
        out_shape=(jax.ShapeDtypeStruct((n, HID), jnp.float32),
                   jax.ShapeDtypeStruct((n, x.shape[1]), jnp.float32)),
    )(partials, conv_bias, bn_gamma, bn_beta, W_lin, b_lin, W_dec, b_dec)

    pb_dec = -(-e_orig // (NW * B))
    e_pad = pb_dec * NW * B
    dsrc = jnp.concatenate([edge_index[0], jnp.zeros((e_pad - e_orig,), jnp.int32)])
    ddst = jnp.concatenate([edge_index[1], jnp.zeros((e_pad - e_orig,), jnp.int32)])
    ab16 = jnp.concatenate([a, b, jnp.zeros((14,), jnp.float32)])
    dec = functools.partial(
        pl.kernel,
        out_type=jax.ShapeDtypeStruct((e_pad,), jnp.float32),
        mesh=mesh,
        compiler_params=pltpu.CompilerParams(needs_layout_passes=False, use_tc_tiling_on_sc=False),
        scratch_types=[
            pltpu.VMEM((pb_dec, B), jnp.int32),
            pltpu.VMEM((pb_dec, B), jnp.int32),
            pltpu.VMEM((B, HID), jnp.float32),
            pltpu.VMEM((B, HID), jnp.float32),
            pltpu.VMEM((B,), jnp.float32),
            pltpu.VMEM((L,), jnp.float32),
            pltpu.SemaphoreType.DMA,
            pltpu.SemaphoreType.DMA,
        ],
    )(functools.partial(_dec_body, pb_dec))
    value1 = dec(z, dsrc.reshape(NW, pb_dec, B), ddst.reshape(NW, pb_dec, B),
                 ab16)[:e_orig]

    return (z, value1, value2)


# speedup vs baseline: 27.0998x; 1.5404x over previous
import functools

import jax
import jax.numpy as jnp
from jax import lax
from jax.experimental import pallas as pl
from jax.experimental.pallas import tpu as pltpu
from jax.experimental.pallas import tpu_sc as plsc

NC = 2
NS = 16
NW = NC * NS
L = 16
B = 128
HEADS = 4
C = 16
HID = HEADS * C
ACC_W = 80


def _proj_body(x_ref, wl_ref, bl_ref, wr_ref, br_ref, xl_ref, xr_ref):
    x = x_ref[...]
    xl_ref[...] = x @ wl_ref[...] + bl_ref[...][None, :]
    xr_ref[...] = x @ wr_ref[...] + br_ref[...][None, :]


def _iota16():
    return lax.iota(jnp.int32, L)


def _msg_body(pb, nacc, xl_hbm, xr_hbm, srcs_hbm, dsts_hbm, att_hbm, part_hbm,
              idx_src, idx_dst, xls, xrd, val, att_v, zrow, acc, sem1, sem2):
    cid = lax.axis_index("c")
    sid = lax.axis_index("s")
    wid = cid * NS + sid
    rows_per_sub = nacc // NS

    zeros16 = jnp.zeros((L,), jnp.float32)
    for c16 in range(ACC_W // L):
        zrow[pl.ds(c16 * L, L)] = zeros16

    def _zero_val(r, carry):
        for c16 in range(ACC_W // L):
            val[r, pl.ds(c16 * L, L)] = zeros16
        return carry
    lax.fori_loop(0, B, _zero_val, 0)

    def _zero_acc(r, carry):
        pltpu.sync_copy(zrow, acc.at[sid * rows_per_sub + r])
        return carry
    lax.fori_loop(0, rows_per_sub, _zero_acc, 0)

    pltpu.sync_copy(srcs_hbm.at[wid], idx_src)
    pltpu.sync_copy(dsts_hbm.at[wid], idx_dst)
    pltpu.sync_copy(att_hbm, att_v)
    plsc.subcore_barrier()

    att_rows = [att_v[h, :] for h in range(HEADS)]
    xls_b = [xls.at[s] for s in range(2)]
    xrd_b = [xrd.at[s] for s in range(2)]
    gsems = [sem1, sem2]

    def _issue(j, s):
        pltpu.async_copy(xl_hbm.at[idx_src.at[j]], xls_b[s], gsems[s])
        pltpu.async_copy(xr_hbm.at[idx_dst.at[j]], xrd_b[s], gsems[s])

    def _wait(j, s):
        pltpu.make_async_copy(xl_hbm.at[idx_src.at[j]], xls_b[s], gsems[s]).wait()
        pltpu.make_async_copy(xr_hbm.at[idx_dst.at[j]], xrd_b[s], gsems[s]).wait()

    _issue(0, 0)
    _issue(1, 1)

    def _block(j2, carry):
        for s in range(2):
            j = j2 * 2 + s
            _wait(j, s)
            xls_s, xrd_s = xls_b[s], xrd_b[s]

            def _group(g, gcarry):
                row = g * L + _iota16()
                for h in range(HEADS):
                    cols = []
                    acc_h = jnp.zeros((L,), jnp.float32)
                    for cc in range(C):
                        col = jnp.full((L,), h * C + cc, jnp.int32)
                        xc = plsc.load_gather(xls_s, [row, col])
                        t = xc + plsc.load_gather(xrd_s, [row, col])
                        lk = jnp.maximum(t, 0.2 * t)
                        acc_h = acc_h + lk * att_rows[h][cc]
                        cols.append(xc)
                    w_h = jnp.exp(acc_h)
                    for cc in range(C):
                        col = jnp.full((L,), h * C + cc, jnp.int32)
                        plsc.store_scatter(val, [row, col], cols[cc] * w_h)
                    plsc.store_scatter(
                        val, [row, jnp.full((L,), HID + h, jnp.int32)], w_h)
                return gcarry
            lax.fori_loop(0, B // L, _group, 0)

            @pl.when(j + 2 < pb)
            def _():
                _issue(j + 2, s)

            pltpu.sync_copy(val, acc.at[idx_dst.at[j]], add=True)
        return carry
    lax.fori_loop(0, pb // 2, _block, 0)

    plsc.subcore_barrier()
    pltpu.sync_copy(acc.at[pl.ds(sid * rows_per_sub, rows_per_sub)],
                    part_hbm.at[cid, pl.ds(sid * rows_per_sub, rows_per_sub)])


def _tail_body(n, part_ref, cb_ref, g_ref, be_ref, wlin_ref, blin_ref,
               wdec_ref, bdec_ref, z_ref, v2_ref):
    P = part_ref[0] + part_ref[1]
    val = P[:n, :HID]
    w16 = P[:n, HID:HID + L]
    r16 = lax.broadcasted_iota(jnp.int32, (L, HID), 0)
    c16 = lax.broadcasted_iota(jnp.int32, (L, HID), 1)
    S = (r16 == c16 // C).astype(jnp.float32)
    den = w16 @ S
    H1 = jnp.maximum(val / den + cb_ref[...][None, :], 0.0)
    mean = jnp.mean(H1, axis=0)
    var = jnp.mean((H1 - mean[None, :]) ** 2, axis=0)
    Hn = (H1 - mean[None, :]) / jnp.sqrt(var + 1e-5) * g_ref[...][None, :] + be_ref[...][None, :]
    z = Hn @ wlin_ref[...] + blin_ref[...][None, :]
    z_ref[...] = z
    v2_ref[...] = z @ wdec_ref[...] + bdec_ref[...][None, :]


def _dec_body(pb, z_hbm, srcs_hbm, dsts_hbm, ab_hbm, v1_hbm,
              idx_src, idx_dst, zs, zd, outb, ab_v, sem1, sem2):
    cid = lax.axis_index("c")
    sid = lax.axis_index("s")
    wid = cid * NS + sid
    pltpu.sync_copy(srcs_hbm.at[wid], idx_src)
    pltpu.sync_copy(dsts_hbm.at[wid], idx_dst)
    pltpu.sync_copy(ab_hbm, ab_v)
    abv = ab_v[...]
    ra = jnp.maximum(abv[0], 0.0)
    sb = abv[1]

    zs_b = [zs.at[s] for s in range(2)]
    zd_b = [zd.at[s] for s in range(2)]
    gsems = [sem1, sem2]

    def _issue(j, s):
        pltpu.async_copy(z_hbm.at[idx_src.at[j]], zs_b[s], gsems[s])
        pltpu.async_copy(z_hbm.at[idx_dst.at[j]], zd_b[s], gsems[s])

    def _wait(j, s):
        pltpu.make_async_copy(z_hbm.at[idx_src.at[j]], zs_b[s], gsems[s]).wait()
        pltpu.make_async_copy(z_hbm.at[idx_dst.at[j]], zd_b[s], gsems[s]).wait()

    _issue(0, 0)
    _issue(1, 1)

    def _block(j2, carry):
        for s in range(2):
            j = j2 * 2 + s
            _wait(j, s)
            zs_s, zd_s = zs_b[s], zd_b[s]

            def _group(g, gcarry):
                row = g * L + _iota16()
                acc = jnp.zeros((L,), jnp.float32)
                for col_i in range(HID):
                    col = jnp.full((L,), col_i, jnp.int32)
                    d = (plsc.load_gather(zs_s, [row, col])
                         - plsc.load_gather(zd_s, [row, col]))
                    acc = acc + d * d
                dist = ra * acc + sb
                outb[pl.ds(g * L, L)] = 1.0 / (1.0 + jnp.exp(dist))
                return gcarry
            lax.fori_loop(0, B // L, _group, 0)

            @pl.when(j + 2 < pb)
            def _():
                _issue(j + 2, s)

            pltpu.sync_copy(outb, v1_hbm.at[pl.ds((wid * pb + j) * B, B)])
        return carry
    lax.fori_loop(0, pb // 2, _block, 0)


def kernel(x, edge_index, edge_attr, W_l, b_l, W_r, b_r, att, conv_bias,
           bn_gamma, bn_beta, W_lin, b_lin, a, b, W_dec, b_dec):
    n = x.shape[0]
    e_orig = edge_index.shape[1]

    xl, xr = pl.pallas_call(
        _proj_body,
        out_shape=(jax.ShapeDtypeStruct((n, HID), jnp.float32),
                   jax.ShapeDtypeStruct((n, HID), jnp.float32)),
    )(x, W_l, b_l, W_r, b_r)

    loop = jnp.arange(n, dtype=edge_index.dtype)
    e2 = e_orig + n
    pb_msg = -(-e2 // (NW * B))
    pb_msg += pb_msg % 2
    e2_pad = pb_msg * NW * B
    src2 = jnp.concatenate([edge_index[0], loop,
                            jnp.zeros((e2_pad - e2,), jnp.int32)])
    dst2 = jnp.concatenate([edge_index[1], loop,
                            jnp.full((e2_pad - e2,), n, jnp.int32)])
    srcs = src2.reshape(NW, pb_msg, B)
    dsts = dst2.reshape(NW, pb_msg, B)
    nacc = -(-(n + 1) // (NS * 8)) * (NS * 8)

    mesh = plsc.VectorSubcoreMesh(core_axis_name="c", subcore_axis_name="s",
                                  num_cores=NC, num_subcores=NS)
    msg = functools.partial(
        pl.kernel,
        out_type=jax.ShapeDtypeStruct((NC, nacc, ACC_W), jnp.float32),
        mesh=mesh,
        compiler_params=pltpu.CompilerParams(needs_layout_passes=False, use_tc_tiling_on_sc=False),
        scratch_types=[
            pltpu.VMEM((pb_msg, B), jnp.int32),
            pltpu.VMEM((pb_msg, B), jnp.int32),
            pltpu.VMEM((2, B, HID), jnp.float32),
            pltpu.VMEM((2, B, HID), jnp.float32),
            pltpu.VMEM((B, ACC_W), jnp.float32),
            pltpu.VMEM((HEADS, C), jnp.float32),
            pltpu.VMEM((ACC_W,), jnp.float32),
            pltpu.VMEM_SHARED((nacc, ACC_W), jnp.float32),
            pltpu.SemaphoreType.DMA,
            pltpu.SemaphoreType.DMA,
        ],
    )(functools.partial(_msg_body, pb_msg, nacc))
    partials = msg(xl, xr, srcs, dsts, att)

    z, value2 = pl.pallas_call(
        functools.partial(_tail_body, n),
        out_shape=(jax.ShapeDtypeStruct((n, HID), jnp.float32),
                   jax.ShapeDtypeStruct((n, x.shape[1]), jnp.float32)),
    )(partials, conv_bias, bn_gamma, bn_beta, W_lin, b_lin, W_dec, b_dec)

    pb_dec = -(-e_orig // (NW * B))
    pb_dec += pb_dec % 2
    e_pad = pb_dec * NW * B
    dsrc = jnp.concatenate([edge_index[0], jnp.zeros((e_pad - e_orig,), jnp.int32)])
    ddst = jnp.concatenate([edge_index[1], jnp.zeros((e_pad - e_orig,), jnp.int32)])
    ab16 = jnp.concatenate([a, b, jnp.zeros((14,), jnp.float32)])
    dec = functools.partial(
        pl.kernel,
        out_type=jax.ShapeDtypeStruct((e_pad,), jnp.float32),
        mesh=mesh,
        compiler_params=pltpu.CompilerParams(needs_layout_passes=False, use_tc_tiling_on_sc=False),
        scratch_types=[
            pltpu.VMEM((pb_dec, B), jnp.int32),
            pltpu.VMEM((pb_dec, B), jnp.int32),
            pltpu.VMEM((2, B, HID), jnp.float32),
            pltpu.VMEM((2, B, HID), jnp.float32),
            pltpu.VMEM((B,), jnp.float32),
            pltpu.VMEM((L,), jnp.float32),
            pltpu.SemaphoreType.DMA,
            pltpu.SemaphoreType.DMA,
        ],
    )(functools.partial(_dec_body, pb_dec))
    value1 = dec(z, dsrc.reshape(NW, pb_dec, B), ddst.reshape(NW, pb_dec, B),
                 ab16)[:e_orig]

    return (z, value1, value2)


# speedup vs baseline: 33.5579x; 1.2383x over previous
import functools

import jax
import jax.numpy as jnp
from jax import lax
from jax.experimental import pallas as pl
from jax.experimental.pallas import tpu as pltpu
from jax.experimental.pallas import tpu_sc as plsc

NC = 2
NS = 16
NW = NC * NS
L = 16
B = 128
HEADS = 4
C = 16
HID = HEADS * C
ACC_W = 80


def _proj_body(x_ref, wl_ref, bl_ref, wr_ref, br_ref, xl_ref, xr_ref):
    x = x_ref[...]
    xl_ref[...] = (x @ wl_ref[...] + bl_ref[...][None, :]).astype(jnp.bfloat16)
    xr_ref[...] = (x @ wr_ref[...] + br_ref[...][None, :]).astype(jnp.bfloat16)


def _iota16():
    return lax.iota(jnp.int32, L)


def _msg_body(pb, nacc, xl_hbm, xr_hbm, srcs_hbm, dsts_hbm, att_hbm, part_hbm,
              idx_src, idx_dst, xls, xrd, val, att_v, zrow, acc,
              sem1, sem2, sem3, sem4):
    cid = lax.axis_index("c")
    sid = lax.axis_index("s")
    wid = cid * NS + sid
    rows_per_sub = nacc // NS

    zeros16 = jnp.zeros((L,), jnp.float32)
    for c16 in range(ACC_W // L):
        zrow[pl.ds(c16 * L, L)] = zeros16

    def _zero_val(r, carry):
        for s in range(2):
            for c16 in range(ACC_W // L):
                val[s, r, pl.ds(c16 * L, L)] = zeros16
        return carry
    lax.fori_loop(0, B, _zero_val, 0)

    def _zero_acc(r, carry):
        pltpu.sync_copy(zrow, acc.at[sid * rows_per_sub + r])
        return carry
    lax.fori_loop(0, rows_per_sub, _zero_acc, 0)

    pltpu.sync_copy(srcs_hbm.at[wid], idx_src)
    pltpu.sync_copy(dsts_hbm.at[wid], idx_dst)
    pltpu.sync_copy(att_hbm, att_v)
    plsc.subcore_barrier()

    att_rows = [att_v[h, :] for h in range(HEADS)]
    xls_b = [xls.at[s] for s in range(2)]
    xrd_b = [xrd.at[s] for s in range(2)]
    val_b = [val.at[s] for s in range(2)]
    gsems = [sem1, sem2]
    vsems = [sem3, sem4]
    WPH = C // 2

    def _issue(j, s):
        pltpu.async_copy(xl_hbm.at[idx_src.at[j]], xls_b[s], gsems[s])
        pltpu.async_copy(xr_hbm.at[idx_dst.at[j]], xrd_b[s], gsems[s])

    def _wait(j, s):
        pltpu.make_async_copy(xl_hbm.at[idx_src.at[j]], xls_b[s], gsems[s]).wait()
        pltpu.make_async_copy(xr_hbm.at[idx_dst.at[j]], xrd_b[s], gsems[s]).wait()

    def _wait_scatter(j, s):
        pltpu.make_async_copy(val_b[s], acc.at[idx_dst.at[j]], vsems[s]).wait()

    _issue(0, 0)
    _issue(1, 1)

    def _block(j2, carry):
        for s in range(2):
            j = j2 * 2 + s
            _wait(j, s)

            @pl.when(j >= 2)
            def _():
                _wait_scatter(j - 2, s)
            xls_s, xrd_s, val_s = xls_b[s], xrd_b[s], val_b[s]

            def _group(g, gcarry):
                row = g * L + _iota16()
                for h in range(HEADS):
                    cols = []
                    acc_h = jnp.zeros((L,), jnp.float32)
                    for k in range(WPH):
                        wcol = jnp.full((L,), h * WPH + k, jnp.int32)
                        le, lo = plsc.unpack(
                            plsc.bitcast(plsc.load_gather(xls_s, [row, wcol]),
                                         jnp.bfloat16),
                            format=plsc.PackFormat.INTERLEAVED,
                            preferred_element_type=jnp.float32)
                        re_, ro = plsc.unpack(
                            plsc.bitcast(plsc.load_gather(xrd_s, [row, wcol]),
                                         jnp.bfloat16),
                            format=plsc.PackFormat.INTERLEAVED,
                            preferred_element_type=jnp.float32)
                        for xc, rc, cc in ((le, re_, 2 * k), (lo, ro, 2 * k + 1)):
                            t = xc + rc
                            lk = jnp.maximum(t, 0.2 * t)
                            acc_h = acc_h + lk * att_rows[h][cc]
                            cols.append(xc)
                    w_h = jnp.exp(acc_h)
                    for cc in range(C):
                        col = jnp.full((L,), h * C + cc, jnp.int32)
                        plsc.store_scatter(val_s, [row, col], cols[cc] * w_h)
                    plsc.store_scatter(
                        val_s, [row, jnp.full((L,), HID + h, jnp.int32)], w_h)
                return gcarry
            lax.fori_loop(0, B // L, _group, 0)

            pltpu.async_copy(val_s, acc.at[idx_dst.at[j]], vsems[s], add=True)

            @pl.when(j + 2 < pb)
            def _():
                _issue(j + 2, s)
        return carry
    lax.fori_loop(0, pb // 2, _block, 0)

    _wait_scatter(pb - 2, 0)
    _wait_scatter(pb - 1, 1)
    plsc.subcore_barrier()
    pltpu.sync_copy(acc.at[pl.ds(sid * rows_per_sub, rows_per_sub)],
                    part_hbm.at[cid, pl.ds(sid * rows_per_sub, rows_per_sub)])


def _tail_body(n, part_ref, cb_ref, g_ref, be_ref, wlin_ref, blin_ref,
               wdec_ref, bdec_ref, z_ref, v2_ref):
    P = part_ref[0] + part_ref[1]
    val = P[:n, :HID]
    w16 = P[:n, HID:HID + L]
    r16 = lax.broadcasted_iota(jnp.int32, (L, HID), 0)
    c16 = lax.broadcasted_iota(jnp.int32, (L, HID), 1)
    S = (r16 == c16 // C).astype(jnp.float32)
    den = w16 @ S
    H1 = jnp.maximum(val / den + cb_ref[...][None, :], 0.0)
    mean = jnp.mean(H1, axis=0)
    var = jnp.mean((H1 - mean[None, :]) ** 2, axis=0)
    Hn = (H1 - mean[None, :]) / jnp.sqrt(var + 1e-5) * g_ref[...][None, :] + be_ref[...][None, :]
    z = Hn @ wlin_ref[...] + blin_ref[...][None, :]
    z_ref[...] = z
    v2_ref[...] = z @ wdec_ref[...] + bdec_ref[...][None, :]


def _dec_body(pb, z_hbm, srcs_hbm, dsts_hbm, ab_hbm, v1_hbm,
              idx_src, idx_dst, zs, zd, outb, ab_v, sem1, sem2):
    cid = lax.axis_index("c")
    sid = lax.axis_index("s")
    wid = cid * NS + sid
    pltpu.sync_copy(srcs_hbm.at[wid], idx_src)
    pltpu.sync_copy(dsts_hbm.at[wid], idx_dst)
    pltpu.sync_copy(ab_hbm, ab_v)
    abv = ab_v[...]
    ra = jnp.maximum(abv[0], 0.0)
    sb = abv[1]

    zs_b = [zs.at[s] for s in range(2)]
    zd_b = [zd.at[s] for s in range(2)]
    gsems = [sem1, sem2]

    def _issue(j, s):
        pltpu.async_copy(z_hbm.at[idx_src.at[j]], zs_b[s], gsems[s])
        pltpu.async_copy(z_hbm.at[idx_dst.at[j]], zd_b[s], gsems[s])

    def _wait(j, s):
        pltpu.make_async_copy(z_hbm.at[idx_src.at[j]], zs_b[s], gsems[s]).wait()
        pltpu.make_async_copy(z_hbm.at[idx_dst.at[j]], zd_b[s], gsems[s]).wait()

    _issue(0, 0)
    _issue(1, 1)

    def _block(j2, carry):
        for s in range(2):
            j = j2 * 2 + s
            _wait(j, s)
            zs_s, zd_s = zs_b[s], zd_b[s]

            def _group(g, gcarry):
                row = g * L + _iota16()
                acc = jnp.zeros((L,), jnp.float32)
                for col_i in range(HID):
                    col = jnp.full((L,), col_i, jnp.int32)
                    d = (plsc.load_gather(zs_s, [row, col])
                         - plsc.load_gather(zd_s, [row, col]))
                    acc = acc + d * d
                dist = ra * acc + sb
                outb[pl.ds(g * L, L)] = 1.0 / (1.0 + jnp.exp(dist))
                return gcarry
            lax.fori_loop(0, B // L, _group, 0)

            @pl.when(j + 2 < pb)
            def _():
                _issue(j + 2, s)

            pltpu.sync_copy(outb, v1_hbm.at[pl.ds((wid * pb + j) * B, B)])
        return carry
    lax.fori_loop(0, pb // 2, _block, 0)


def kernel(x, edge_index, edge_attr, W_l, b_l, W_r, b_r, att, conv_bias,
           bn_gamma, bn_beta, W_lin, b_lin, a, b, W_dec, b_dec):
    n = x.shape[0]
    e_orig = edge_index.shape[1]

    xl_bf, xr_bf = pl.pallas_call(
        _proj_body,
        out_shape=(jax.ShapeDtypeStruct((n, HID), jnp.bfloat16),
                   jax.ShapeDtypeStruct((n, HID), jnp.bfloat16)),
    )(x, W_l, b_l, W_r, b_r)
    xlp = lax.bitcast_convert_type(xl_bf.reshape(n, HID // 2, 2), jnp.int32)
    xrp = lax.bitcast_convert_type(xr_bf.reshape(n, HID // 2, 2), jnp.int32)

    loop = jnp.arange(n, dtype=edge_index.dtype)
    e2 = e_orig + n
    pb_msg = -(-e2 // (NW * B))
    pb_msg += pb_msg % 2
    e2_pad = pb_msg * NW * B
    src2 = jnp.concatenate([edge_index[0], loop,
                            jnp.zeros((e2_pad - e2,), jnp.int32)])
    dst2 = jnp.concatenate([edge_index[1], loop,
                            jnp.full((e2_pad - e2,), n, jnp.int32)])
    srcs = src2.reshape(NW, pb_msg, B)
    dsts = dst2.reshape(NW, pb_msg, B)
    nacc = -(-(n + 1) // (NS * 8)) * (NS * 8)

    mesh = plsc.VectorSubcoreMesh(core_axis_name="c", subcore_axis_name="s",
                                  num_cores=NC, num_subcores=NS)
    msg = functools.partial(
        pl.kernel,
        out_type=jax.ShapeDtypeStruct((NC, nacc, ACC_W), jnp.float32),
        mesh=mesh,
        compiler_params=pltpu.CompilerParams(needs_layout_passes=False, use_tc_tiling_on_sc=False),
        scratch_types=[
            pltpu.VMEM((pb_msg, B), jnp.int32),
            pltpu.VMEM((pb_msg, B), jnp.int32),
            pltpu.VMEM((2, B, HID // 2), jnp.int32),
            pltpu.VMEM((2, B, HID // 2), jnp.int32),
            pltpu.VMEM((2, B, ACC_W), jnp.float32),
            pltpu.VMEM((HEADS, C), jnp.float32),
            pltpu.VMEM((ACC_W,), jnp.float32),
            pltpu.VMEM_SHARED((nacc, ACC_W), jnp.float32),
            pltpu.SemaphoreType.DMA,
            pltpu.SemaphoreType.DMA,
            pltpu.SemaphoreType.DMA,
            pltpu.SemaphoreType.DMA,
        ],
    )(functools.partial(_msg_body, pb_msg, nacc))
    partials = msg(xlp, xrp, srcs, dsts, att)

    z, value2 = pl.pallas_call(
        functools.partial(_tail_body, n),
        out_shape=(jax.ShapeDtypeStruct((n, HID), jnp.float32),
                   jax.ShapeDtypeStruct((n, x.shape[1]), jnp.float32)),
    )(partials, conv_bias, bn_gamma, bn_beta, W_lin, b_lin, W_dec, b_dec)

    pb_dec = -(-e_orig // (NW * B))
    pb_dec += pb_dec % 2
    e_pad = pb_dec * NW * B
    dsrc = jnp.concatenate([edge_index[0], jnp.zeros((e_pad - e_orig,), jnp.int32)])
    ddst = jnp.concatenate([edge_index[1], jnp.zeros((e_pad - e_orig,), jnp.int32)])
    ab16 = jnp.concatenate([a, b, jnp.zeros((14,), jnp.float32)])
    dec = functools.partial(
        pl.kernel,
        out_type=jax.ShapeDtypeStruct((e_pad,), jnp.float32),
        mesh=mesh,
        compiler_params=pltpu.CompilerParams(needs_layout_passes=False, use_tc_tiling_on_sc=False),
        scratch_types=[
            pltpu.VMEM((pb_dec, B), jnp.int32),
            pltpu.VMEM((pb_dec, B), jnp.int32),
            pltpu.VMEM((2, B, HID), jnp.float32),
            pltpu.VMEM((2, B, HID), jnp.float32),
            pltpu.VMEM((B,), jnp.float32),
            pltpu.VMEM((L,), jnp.float32),
            pltpu.SemaphoreType.DMA,
            pltpu.SemaphoreType.DMA,
        ],
    )(functools.partial(_dec_body, pb_dec))
    value1 = dec(z, dsrc.reshape(NW, pb_dec, B), ddst.reshape(NW, pb_dec, B),
                 ab16)[:e_orig]

    return (z, value1, value2)


# speedup vs baseline: 43.4806x; 1.2957x over previous
import functools

import jax
import jax.numpy as jnp
from jax import lax
from jax.experimental import pallas as pl
from jax.experimental.pallas import tpu as pltpu
from jax.experimental.pallas import tpu_sc as plsc

NC = 2
NS = 16
NW = NC * NS
L = 16
B = 128
HEADS = 4
C = 16
HID = HEADS * C
ACC_W = 80


def _proj_body(x_ref, wl_ref, bl_ref, wr_ref, br_ref, xl_ref, xr_ref):
    x = x_ref[...]
    xl_ref[...] = (x @ wl_ref[...] + bl_ref[...][None, :]).astype(jnp.bfloat16)
    xr_ref[...] = (x @ wr_ref[...] + br_ref[...][None, :]).astype(jnp.bfloat16)


def _iota16():
    return lax.iota(jnp.int32, L)


def _msg_body(pb, nacc, xl_hbm, xr_hbm, srcs_hbm, dsts_hbm, att_hbm, part_hbm,
              idx_src, idx_dst, xls, xrd, val, att_v, zrow, acc,
              sem1, sem2, sem3, sem4):
    cid = lax.axis_index("c")
    sid = lax.axis_index("s")
    wid = cid * NS + sid
    rows_per_sub = nacc // NS

    zeros16 = jnp.zeros((L,), jnp.float32)
    for c16 in range(ACC_W // L):
        zrow[pl.ds(c16 * L, L)] = zeros16

    def _zero_val(r, carry):
        for s in range(2):
            for c16 in range(ACC_W // L):
                val[s, r, pl.ds(c16 * L, L)] = zeros16
        return carry
    lax.fori_loop(0, B, _zero_val, 0)

    def _zero_acc(r, carry):
        pltpu.sync_copy(zrow, acc.at[sid * rows_per_sub + r])
        return carry
    lax.fori_loop(0, rows_per_sub, _zero_acc, 0)

    pltpu.sync_copy(srcs_hbm.at[wid], idx_src)
    pltpu.sync_copy(dsts_hbm.at[wid], idx_dst)
    pltpu.sync_copy(att_hbm, att_v)
    plsc.subcore_barrier()

    att_rows = [att_v[h, :] for h in range(HEADS)]
    xls_b = [xls.at[s] for s in range(2)]
    xrd_b = [xrd.at[s] for s in range(2)]
    val_b = [val.at[s] for s in range(2)]
    gsems = [sem1, sem2]
    vsems = [sem3, sem4]
    WPH = C // 2

    def _issue(j, s):
        pltpu.async_copy(xl_hbm.at[idx_src.at[j]], xls_b[s], gsems[s])
        pltpu.async_copy(xr_hbm.at[idx_dst.at[j]], xrd_b[s], gsems[s])

    def _wait(j, s):
        pltpu.make_async_copy(xl_hbm.at[idx_src.at[j]], xls_b[s], gsems[s]).wait()
        pltpu.make_async_copy(xr_hbm.at[idx_dst.at[j]], xrd_b[s], gsems[s]).wait()

    def _wait_scatter(j, s):
        pltpu.make_async_copy(val_b[s], acc.at[idx_dst.at[j]], vsems[s]).wait()

    _issue(0, 0)
    _issue(1, 1)

    def _block(j2, carry):
        for s in range(2):
            j = j2 * 2 + s
            _wait(j, s)

            @pl.when(j >= 2)
            def _():
                _wait_scatter(j - 2, s)
            xls_s, xrd_s, val_s = xls_b[s], xrd_b[s], val_b[s]

            def _group(g, gcarry):
                row = g * L + _iota16()
                for h in range(HEADS):
                    cols = []
                    acc_h = jnp.zeros((L,), jnp.float32)
                    for k in range(WPH):
                        wcol = jnp.full((L,), h * WPH + k, jnp.int32)
                        le, lo = plsc.unpack(
                            plsc.bitcast(plsc.load_gather(xls_s, [row, wcol]),
                                         jnp.bfloat16),
                            format=plsc.PackFormat.INTERLEAVED,
                            preferred_element_type=jnp.float32)
                        re_, ro = plsc.unpack(
                            plsc.bitcast(plsc.load_gather(xrd_s, [row, wcol]),
                                         jnp.bfloat16),
                            format=plsc.PackFormat.INTERLEAVED,
                            preferred_element_type=jnp.float32)
                        for xc, rc, cc in ((le, re_, 2 * k), (lo, ro, 2 * k + 1)):
                            t = xc + rc
                            lk = jnp.maximum(t, 0.2 * t)
                            acc_h = acc_h + lk * att_rows[h][cc]
                            cols.append(xc)
                    w_h = jnp.exp(acc_h)
                    for cc in range(C):
                        col = jnp.full((L,), h * C + cc, jnp.int32)
                        plsc.store_scatter(val_s, [row, col], cols[cc] * w_h)
                    plsc.store_scatter(
                        val_s, [row, jnp.full((L,), HID + h, jnp.int32)], w_h)
                return gcarry
            lax.fori_loop(0, B // L, _group, 0)

            pltpu.async_copy(val_s, acc.at[idx_dst.at[j]], vsems[s], add=True)

            @pl.when(j + 2 < pb)
            def _():
                _issue(j + 2, s)
        return carry
    lax.fori_loop(0, pb // 2, _block, 0)

    _wait_scatter(pb - 2, 0)
    _wait_scatter(pb - 1, 1)
    plsc.subcore_barrier()
    pltpu.sync_copy(acc.at[pl.ds(sid * rows_per_sub, rows_per_sub)],
                    part_hbm.at[cid, pl.ds(sid * rows_per_sub, rows_per_sub)])


def _tail_body(n, part_ref, cb_ref, g_ref, be_ref, wlin_ref, blin_ref,
               wdec_ref, bdec_ref, z_ref, v2_ref):
    P = part_ref[0] + part_ref[1]
    val = P[:n, :HID]
    w16 = P[:n, HID:HID + L]
    r16 = lax.broadcasted_iota(jnp.int32, (L, HID), 0)
    c16 = lax.broadcasted_iota(jnp.int32, (L, HID), 1)
    S = (r16 == c16 // C).astype(jnp.float32)
    den = w16 @ S
    H1 = jnp.maximum(val / den + cb_ref[...][None, :], 0.0)
    mean = jnp.mean(H1, axis=0)
    var = jnp.mean((H1 - mean[None, :]) ** 2, axis=0)
    Hn = (H1 - mean[None, :]) / jnp.sqrt(var + 1e-5) * g_ref[...][None, :] + be_ref[...][None, :]
    z = Hn @ wlin_ref[...] + blin_ref[...][None, :]
    z_ref[...] = z
    v2_ref[...] = z @ wdec_ref[...] + bdec_ref[...][None, :]


def _dec_body(pb, z_hbm, srcs_hbm, dsts_hbm, ab_hbm, v1_hbm,
              idx_src, idx_dst, zs, zd, outb, ab_v, sem1, sem2, sem3, sem4):
    cid = lax.axis_index("c")
    sid = lax.axis_index("s")
    wid = cid * NS + sid
    pltpu.sync_copy(srcs_hbm.at[wid], idx_src)
    pltpu.sync_copy(dsts_hbm.at[wid], idx_dst)
    pltpu.sync_copy(ab_hbm, ab_v)
    abv = ab_v[...]
    ra = jnp.maximum(abv[0], 0.0)
    sb = abv[1]

    zs_b = [zs.at[s] for s in range(2)]
    zd_b = [zd.at[s] for s in range(2)]
    outb_b = [outb.at[s] for s in range(2)]
    gsems = [sem1, sem2]
    osems = [sem3, sem4]

    def _issue(j, s):
        pltpu.async_copy(z_hbm.at[idx_src.at[j]], zs_b[s], gsems[s])
        pltpu.async_copy(z_hbm.at[idx_dst.at[j]], zd_b[s], gsems[s])

    def _wait(j, s):
        pltpu.make_async_copy(z_hbm.at[idx_src.at[j]], zs_b[s], gsems[s]).wait()
        pltpu.make_async_copy(z_hbm.at[idx_dst.at[j]], zd_b[s], gsems[s]).wait()

    def _wait_store(j, s):
        pltpu.make_async_copy(outb_b[s],
                              v1_hbm.at[pl.ds((wid * pb + j) * B, B)],
                              osems[s]).wait()

    _issue(0, 0)
    _issue(1, 1)

    def _block(j2, carry):
        for s in range(2):
            j = j2 * 2 + s
            _wait(j, s)

            @pl.when(j >= 2)
            def _():
                _wait_store(j - 2, s)
            zs_s, zd_s, outb_s = zs_b[s], zd_b[s], outb_b[s]

            def _group(g, gcarry):
                row = g * L + _iota16()
                acc = jnp.zeros((L,), jnp.float32)
                for k in range(HID // 2):
                    wcol = jnp.full((L,), k, jnp.int32)
                    se, so = plsc.unpack(
                        plsc.bitcast(plsc.load_gather(zs_s, [row, wcol]),
                                     jnp.bfloat16),
                        format=plsc.PackFormat.INTERLEAVED,
                        preferred_element_type=jnp.float32)
                    de, do = plsc.unpack(
                        plsc.bitcast(plsc.load_gather(zd_s, [row, wcol]),
                                     jnp.bfloat16),
                        format=plsc.PackFormat.INTERLEAVED,
                        preferred_element_type=jnp.float32)
                    d0 = se - de
                    d1 = so - do
                    acc = acc + d0 * d0 + d1 * d1
                dist = ra * acc + sb
                outb_s[pl.ds(g * L, L)] = 1.0 / (1.0 + jnp.exp(dist))
                return gcarry
            lax.fori_loop(0, B // L, _group, 0)

            pltpu.async_copy(outb_s, v1_hbm.at[pl.ds((wid * pb + j) * B, B)],
                             osems[s])

            @pl.when(j + 2 < pb)
            def _():
                _issue(j + 2, s)
        return carry
    lax.fori_loop(0, pb // 2, _block, 0)

    _wait_store(pb - 2, 0)
    _wait_store(pb - 1, 1)


def kernel(x, edge_index, edge_attr, W_l, b_l, W_r, b_r, att, conv_bias,
           bn_gamma, bn_beta, W_lin, b_lin, a, b, W_dec, b_dec):
    n = x.shape[0]
    e_orig = edge_index.shape[1]

    xl_bf, xr_bf = pl.pallas_call(
        _proj_body,
        out_shape=(jax.ShapeDtypeStruct((n, HID), jnp.bfloat16),
                   jax.ShapeDtypeStruct((n, HID), jnp.bfloat16)),
    )(x, W_l, b_l, W_r, b_r)
    xlp = lax.bitcast_convert_type(xl_bf.reshape(n, HID // 2, 2), jnp.int32)
    xrp = lax.bitcast_convert_type(xr_bf.reshape(n, HID // 2, 2), jnp.int32)

    loop = jnp.arange(n, dtype=edge_index.dtype)
    e2 = e_orig + n
    pb_msg = -(-e2 // (NW * B))
    pb_msg += pb_msg % 2
    e2_pad = pb_msg * NW * B
    src2 = jnp.concatenate([edge_index[0], loop,
                            jnp.zeros((e2_pad - e2,), jnp.int32)])
    dst2 = jnp.concatenate([edge_index[1], loop,
                            jnp.full((e2_pad - e2,), n, jnp.int32)])
    srcs = src2.reshape(NW, pb_msg, B)
    dsts = dst2.reshape(NW, pb_msg, B)
    nacc = -(-(n + 1) // (NS * 8)) * (NS * 8)

    mesh = plsc.VectorSubcoreMesh(core_axis_name="c", subcore_axis_name="s",
                                  num_cores=NC, num_subcores=NS)
    msg = functools.partial(
        pl.kernel,
        out_type=jax.ShapeDtypeStruct((NC, nacc, ACC_W), jnp.float32),
        mesh=mesh,
        compiler_params=pltpu.CompilerParams(needs_layout_passes=False, use_tc_tiling_on_sc=False),
        scratch_types=[
            pltpu.VMEM((pb_msg, B), jnp.int32),
            pltpu.VMEM((pb_msg, B), jnp.int32),
            pltpu.VMEM((2, B, HID // 2), jnp.int32),
            pltpu.VMEM((2, B, HID // 2), jnp.int32),
            pltpu.VMEM((2, B, ACC_W), jnp.float32),
            pltpu.VMEM((HEADS, C), jnp.float32),
            pltpu.VMEM((ACC_W,), jnp.float32),
            pltpu.VMEM_SHARED((nacc, ACC_W), jnp.float32),
            pltpu.SemaphoreType.DMA,
            pltpu.SemaphoreType.DMA,
            pltpu.SemaphoreType.DMA,
            pltpu.SemaphoreType.DMA,
        ],
    )(functools.partial(_msg_body, pb_msg, nacc))
    partials = msg(xlp, xrp, srcs, dsts, att)

    z, value2 = pl.pallas_call(
        functools.partial(_tail_body, n),
        out_shape=(jax.ShapeDtypeStruct((n, HID), jnp.float32),
                   jax.ShapeDtypeStruct((n, x.shape[1]), jnp.float32)),
    )(partials, conv_bias, bn_gamma, bn_beta, W_lin, b_lin, W_dec, b_dec)

    pb_dec = -(-e_orig // (NW * B))
    pb_dec += pb_dec % 2
    e_pad = pb_dec * NW * B
    dsrc = jnp.concatenate([edge_index[0], jnp.zeros((e_pad - e_orig,), jnp.int32)])
    ddst = jnp.concatenate([edge_index[1], jnp.zeros((e_pad - e_orig,), jnp.int32)])
    ab16 = jnp.concatenate([a, b, jnp.zeros((14,), jnp.float32)])
    dec = functools.partial(
        pl.kernel,
        out_type=jax.ShapeDtypeStruct((e_pad,), jnp.float32),
        mesh=mesh,
        compiler_params=pltpu.CompilerParams(needs_layout_passes=False, use_tc_tiling_on_sc=False),
        scratch_types=[
            pltpu.VMEM((pb_dec, B), jnp.int32),
            pltpu.VMEM((pb_dec, B), jnp.int32),
            pltpu.VMEM((2, B, HID // 2), jnp.int32),
            pltpu.VMEM((2, B, HID // 2), jnp.int32),
            pltpu.VMEM((2, B), jnp.float32),
            pltpu.VMEM((L,), jnp.float32),
            pltpu.SemaphoreType.DMA,
            pltpu.SemaphoreType.DMA,
            pltpu.SemaphoreType.DMA,
            pltpu.SemaphoreType.DMA,
        ],
    )(functools.partial(_dec_body, pb_dec))
    zp = lax.bitcast_convert_type(
        z.astype(jnp.bfloat16).reshape(n, HID // 2, 2), jnp.int32)
    value1 = dec(zp, dsrc.reshape(NW, pb_dec, B), ddst.reshape(NW, pb_dec, B),
                 ab16)[:e_orig]

    return (z, value1, value2)
